# baseline (speedup 1.0000x reference)
import functools

import jax
import jax.numpy as jnp
from jax import lax
from jax.experimental import pallas as pl
from jax.experimental.pallas import tpu as pltpu
from jax.experimental.pallas import tpu_sc as plsc

N = 10000
E = 320000
H = 128

NC = 2
NS = 16
NW = NC * NS
EPW = E // NW
CHUNK = 40
NCHUNKS = EPW // CHUNK
NPAD = 10240
NRPS = NPAD // NS
NB = 4
NGROUPS = NCHUNKS // NB


def _agg_entry(table, src3, dst3, zeros, out, *rest):
    idxs = rest[0:NB]
    idxd = rest[NB:2 * NB]
    rowbufs = rest[2 * NB:3 * NB]
    acc = rest[3 * NB]
    isems_s = rest[3 * NB + 1:4 * NB + 1]
    isems_d = rest[4 * NB + 1:5 * NB + 1]
    gsems = rest[5 * NB + 1:6 * NB + 1]
    ssems = rest[6 * NB + 1:7 * NB + 1]

    cid = lax.axis_index("c")
    sid = lax.axis_index("s")
    wid = sid * NC + cid
    r0 = sid * NRPS

    @pl.when(jnp.logical_and(cid == 0, sid < NS - 1))
    def _():
        pltpu.sync_copy(table.at[pl.ds(r0, NRPS)], acc.at[pl.ds(r0, NRPS)])

    @pl.when(jnp.logical_and(cid == 0, sid == NS - 1))
    def _():
        last = NS - 1
        pltpu.sync_copy(table.at[pl.ds(last * NRPS, N - last * NRPS)],
                        acc.at[pl.ds(last * NRPS, N - last * NRPS)])
        pltpu.sync_copy(zeros.at[pl.ds(N, NPAD - N)], acc.at[pl.ds(N, NPAD - N)])

    @pl.when(cid != 0)
    def _():
        pltpu.sync_copy(zeros.at[pl.ds(r0, NRPS)], acc.at[pl.ds(r0, NRPS)])

    plsc.subcore_barrier()

    def _wait_prev_scatter(b):
        pltpu.make_async_copy(rowbufs[b], acc.at[idxd[b]], ssems[b]).wait()

    def _issue_idx(base, b):
        off = base + b * CHUNK
        pltpu.async_copy(src3.at[pl.ds(off, CHUNK)], idxs[b], isems_s[b])
        pltpu.async_copy(dst3.at[pl.ds(off, CHUNK)], idxd[b], isems_d[b])

    def _issue_gather(b):
        pltpu.make_async_copy(src3.at[pl.ds(0, CHUNK)], idxs[b],
                              isems_s[b]).wait()
        pltpu.async_copy(table.at[idxs[b]], rowbufs[b], gsems[b])

    def _issue_scatter(b):
        pltpu.make_async_copy(table.at[idxs[b]], rowbufs[b], gsems[b]).wait()
        pltpu.make_async_copy(dst3.at[pl.ds(0, CHUNK)], idxd[b],
                              isems_d[b]).wait()
        pltpu.async_copy(rowbufs[b], acc.at[idxd[b]], ssems[b], add=True)

    def group(q, carry):
        base = wid * EPW + q * NB * CHUNK
        for b in (0, 1):
            @pl.when(q > 0)
            def _(b=b):
                _wait_prev_scatter(b)
            _issue_idx(base, b)
        for b in (0, 1):
            _issue_gather(b)
        for b in (2, 3):
            @pl.when(q > 0)
            def _(b=b):
                _wait_prev_scatter(b)
            _issue_idx(base, b)
        for b in (0, 1):
            _issue_scatter(b)
        for b in (2, 3):
            _issue_gather(b)
        for b in (2, 3):
            _issue_scatter(b)
        return carry

    lax.fori_loop(0, NGROUPS, group, 0)
    for b in range(NB):
        _wait_prev_scatter(b)

    for j in range(NB * NGROUPS, NCHUNKS):
        off = wid * EPW + j * CHUNK
        pltpu.sync_copy(src3.at[pl.ds(off, CHUNK)], idxs[0])
        pltpu.sync_copy(dst3.at[pl.ds(off, CHUNK)], idxd[0])
        pltpu.async_copy(table.at[idxs[0]], rowbufs[0], gsems[0]).wait()
        pltpu.sync_copy(rowbufs[0], acc.at[idxd[0]], add=True)

    plsc.subcore_barrier()
    pltpu.sync_copy(acc.at[pl.ds(r0, NRPS)], out.at[cid, pl.ds(r0, NRPS)])


_agg = pl.kernel(
    _agg_entry,
    out_type=jax.ShapeDtypeStruct((NC, NPAD, H), jnp.float32),
    mesh=plsc.VectorSubcoreMesh(core_axis_name="c", subcore_axis_name="s"),
    scratch_types=(
        [pltpu.VMEM((CHUNK,), jnp.int32) for _ in range(2 * NB)]
        + [pltpu.VMEM((CHUNK, H), jnp.float32) for _ in range(NB)]
        + [pltpu.VMEM_SHARED((NPAD, H), jnp.float32)]
        + [pltpu.SemaphoreType.DMA for _ in range(4 * NB)]
    ),
)


def _mlp_body(p_ref, w_ref, b_ref, o_ref, *, relu):
    z = p_ref[0] + p_ref[1]
    y = jnp.dot(z, w_ref[...], preferred_element_type=jnp.float32) + b_ref[...]
    if relu:
        y = jnp.maximum(y, 0.0)
    o_ref[...] = y


def _mlp(p, w, b, relu):
    bn = 2000
    return pl.pallas_call(
        functools.partial(_mlp_body, relu=relu),
        grid=(N // bn,),
        in_specs=[
            pl.BlockSpec((NC, bn, H), lambda i: (0, i, 0)),
            pl.BlockSpec((H, H), lambda i: (0, 0)),
            pl.BlockSpec((1, H), lambda i: (0, 0)),
        ],
        out_specs=pl.BlockSpec((bn, H), lambda i: (i, 0)),
        out_shape=jax.ShapeDtypeStruct((N, H), jnp.float32),
    )(p, w, b.reshape(1, H))


def kernel(x, edge_index, W0, b0, bn_gamma, bn_beta, W1, b1):
    src = edge_index[0]
    dst = edge_index[1]
    zeros = jnp.zeros((NPAD, H), jnp.float32)
    s = bn_gamma * jax.lax.rsqrt(1.0 + 1e-5)
    W0p = W0.T * s[None, :]
    b0p = b0 * s + bn_beta

    p0 = _agg(x, src, dst, zeros)
    h = _mlp(p0, W0p, b0p, relu=True)
    p1 = _agg(h, src, dst, zeros)
    out = _mlp(p1, W1.T, b1, relu=False)
    return out


# speedup vs baseline: 1.1928x; 1.1928x over previous
import functools

import jax
import jax.numpy as jnp
from jax import lax
from jax.experimental import pallas as pl
from jax.experimental.pallas import tpu as pltpu
from jax.experimental.pallas import tpu_sc as plsc

N = 10000
E = 320000
H = 128

NC = 2
NS = 16
NW = NC * NS
EPW = E // NW
CHUNK = 128
NCHUNKS = EPW // CHUNK
ETAIL = EPW - NCHUNKS * CHUNK
NB = 3
NGROUPS = NCHUNKS // NB
RPS = 624
RPS_LAST = N - (NS - 1) * RPS


def _agg_entry(table, src3, dst3, zeros, out, *rest):
    idxs = rest[0:NB]
    idxd = rest[NB:2 * NB]
    rowbufs = rest[2 * NB:3 * NB]
    tidx_s, tidx_d = rest[3 * NB], rest[3 * NB + 1]
    acc = rest[3 * NB + 2]
    base_s = 3 * NB + 3
    isems_s = rest[base_s:base_s + NB]
    isems_d = rest[base_s + NB:base_s + 2 * NB]
    gsems = rest[base_s + 2 * NB:base_s + 3 * NB]
    ssems = rest[base_s + 3 * NB:base_s + 4 * NB]

    cid = lax.axis_index("c")
    sid = lax.axis_index("s")
    wid = sid * NC + cid
    r0 = sid * RPS

    @pl.when(jnp.logical_and(cid == 0, sid < NS - 1))
    def _():
        pltpu.sync_copy(table.at[pl.ds(r0, RPS)], acc.at[pl.ds(r0, RPS)])

    @pl.when(jnp.logical_and(cid == 0, sid == NS - 1))
    def _():
        last = (NS - 1) * RPS
        pltpu.sync_copy(table.at[pl.ds(last, RPS_LAST)],
                        acc.at[pl.ds(last, RPS_LAST)])

    @pl.when(jnp.logical_and(cid != 0, sid < NS - 1))
    def _():
        pltpu.sync_copy(zeros.at[pl.ds(r0, RPS)], acc.at[pl.ds(r0, RPS)])

    @pl.when(jnp.logical_and(cid != 0, sid == NS - 1))
    def _():
        last = (NS - 1) * RPS
        pltpu.sync_copy(zeros.at[pl.ds(last, RPS_LAST)],
                        acc.at[pl.ds(last, RPS_LAST)])

    plsc.subcore_barrier()

    def _wait_prev_scatter(b):
        pltpu.make_async_copy(rowbufs[b], acc.at[idxd[b]], ssems[b]).wait()

    def group(q, carry):
        base = wid * EPW + q * NB * CHUNK
        for b in range(NB):
            @pl.when(q > 0)
            def _(b=b):
                _wait_prev_scatter(b)
            off = base + b * CHUNK
            pltpu.async_copy(src3.at[pl.ds(off, CHUNK)], idxs[b], isems_s[b])
            pltpu.async_copy(dst3.at[pl.ds(off, CHUNK)], idxd[b], isems_d[b])
        for b in range(NB):
            pltpu.make_async_copy(src3.at[pl.ds(0, CHUNK)], idxs[b],
                                  isems_s[b]).wait()
            pltpu.async_copy(table.at[idxs[b]], rowbufs[b], gsems[b])
        for b in range(NB):
            pltpu.make_async_copy(table.at[idxs[b]], rowbufs[b],
                                  gsems[b]).wait()
            pltpu.make_async_copy(dst3.at[pl.ds(0, CHUNK)], idxd[b],
                                  isems_d[b]).wait()
            pltpu.async_copy(rowbufs[b], acc.at[idxd[b]], ssems[b], add=True)
        return carry

    lax.fori_loop(0, NGROUPS, group, 0)
    for b in range(NB):
        _wait_prev_scatter(b)

    if ETAIL:
        toff = wid * EPW + NCHUNKS * CHUNK
        pltpu.sync_copy(src3.at[pl.ds(toff, ETAIL)], tidx_s)
        pltpu.sync_copy(dst3.at[pl.ds(toff, ETAIL)], tidx_d)
        pltpu.async_copy(table.at[tidx_s], rowbufs[0].at[pl.ds(0, ETAIL)],
                         gsems[0]).wait()
        pltpu.sync_copy(rowbufs[0].at[pl.ds(0, ETAIL)], acc.at[tidx_d],
                        add=True)

    plsc.subcore_barrier()

    @pl.when(sid < NS - 1)
    def _():
        pltpu.sync_copy(acc.at[pl.ds(r0, RPS)], out.at[cid, pl.ds(r0, RPS)])

    @pl.when(sid == NS - 1)
    def _():
        last = (NS - 1) * RPS
        pltpu.sync_copy(acc.at[pl.ds(last, RPS_LAST)],
                        out.at[cid, pl.ds(last, RPS_LAST)])


_agg = pl.kernel(
    _agg_entry,
    out_type=jax.ShapeDtypeStruct((NC, N, H), jnp.float32),
    mesh=plsc.VectorSubcoreMesh(core_axis_name="c", subcore_axis_name="s"),
    scratch_types=(
        [pltpu.VMEM((CHUNK,), jnp.int32) for _ in range(2 * NB)]
        + [pltpu.VMEM((CHUNK, H), jnp.float32) for _ in range(NB)]
        + [pltpu.VMEM((ETAIL,), jnp.int32) for _ in range(2)]
        + [pltpu.VMEM_SHARED((N, H), jnp.float32)]
        + [pltpu.SemaphoreType.DMA for _ in range(4 * NB)]
    ),
)


def _mlp_body(p_ref, w_ref, b_ref, o_ref, *, relu):
    z = p_ref[0] + p_ref[1]
    y = jnp.dot(z, w_ref[...], preferred_element_type=jnp.float32) + b_ref[...]
    if relu:
        y = jnp.maximum(y, 0.0)
    o_ref[...] = y


def _mlp(p, w, b, relu):
    bn = 2000
    return pl.pallas_call(
        functools.partial(_mlp_body, relu=relu),
        grid=(N // bn,),
        in_specs=[
            pl.BlockSpec((NC, bn, H), lambda i: (0, i, 0)),
            pl.BlockSpec((H, H), lambda i: (0, 0)),
            pl.BlockSpec((1, H), lambda i: (0, 0)),
        ],
        out_specs=pl.BlockSpec((bn, H), lambda i: (i, 0)),
        out_shape=jax.ShapeDtypeStruct((N, H), jnp.float32),
    )(p, w, b.reshape(1, H))


def kernel(x, edge_index, W0, b0, bn_gamma, bn_beta, W1, b1):
    src = edge_index[0]
    dst = edge_index[1]
    zeros = jnp.zeros((N, H), jnp.float32)
    s = bn_gamma * jax.lax.rsqrt(1.0 + 1e-5)
    W0p = W0.T * s[None, :]
    b0p = b0 * s + bn_beta

    p0 = _agg(x, src, dst, zeros)
    h = _mlp(p0, W0p, b0p, relu=True)
    p1 = _agg(h, src, dst, zeros)
    out = _mlp(p1, W1.T, b1, relu=False)
    return out


# speedup vs baseline: 1.2993x; 1.0893x over previous
import functools

import jax
import jax.numpy as jnp
from jax import lax
from jax.experimental import pallas as pl
from jax.experimental.pallas import tpu as pltpu
from jax.experimental.pallas import tpu_sc as plsc

N = 10000
E = 320000
H = 128

NC = 2
NS = 16
NW = NC * NS
EPW = E // NW
CHUNK = 96
NCHUNKS = EPW // CHUNK
ETAIL = EPW - NCHUNKS * CHUNK
NB = 4
NGROUPS = NCHUNKS // NB
RPS = 624
RPS_LAST = N - (NS - 1) * RPS


def _agg_entry(table, src3, dst3, zeros, out, *rest):
    idxs = rest[0:NB]
    idxd = rest[NB:2 * NB]
    rowbufs = rest[2 * NB:3 * NB]
    tidx_s, tidx_d = rest[3 * NB], rest[3 * NB + 1]
    acc = rest[3 * NB + 2]
    base_s = 3 * NB + 3
    isems_s = rest[base_s:base_s + NB]
    isems_d = rest[base_s + NB:base_s + 2 * NB]
    gsems = rest[base_s + 2 * NB:base_s + 3 * NB]
    ssems = rest[base_s + 3 * NB:base_s + 4 * NB]

    cid = lax.axis_index("c")
    sid = lax.axis_index("s")
    wid = sid * NC + cid
    r0 = sid * RPS

    @pl.when(jnp.logical_and(cid == 0, sid < NS - 1))
    def _():
        pltpu.sync_copy(table.at[pl.ds(r0, RPS)], acc.at[pl.ds(r0, RPS)])

    @pl.when(jnp.logical_and(cid == 0, sid == NS - 1))
    def _():
        last = (NS - 1) * RPS
        pltpu.sync_copy(table.at[pl.ds(last, RPS_LAST)],
                        acc.at[pl.ds(last, RPS_LAST)])

    @pl.when(jnp.logical_and(cid != 0, sid < NS - 1))
    def _():
        pltpu.sync_copy(zeros.at[pl.ds(r0, RPS)], acc.at[pl.ds(r0, RPS)])

    @pl.when(jnp.logical_and(cid != 0, sid == NS - 1))
    def _():
        last = (NS - 1) * RPS
        pltpu.sync_copy(zeros.at[pl.ds(last, RPS_LAST)],
                        acc.at[pl.ds(last, RPS_LAST)])

    plsc.subcore_barrier()

    def _wait_prev_scatter(b):
        pltpu.make_async_copy(rowbufs[b], acc.at[idxd[b]], ssems[b]).wait()

    def _issue_idx(base, b):
        off = base + b * CHUNK
        pltpu.async_copy(src3.at[pl.ds(off, CHUNK)], idxs[b], isems_s[b])
        pltpu.async_copy(dst3.at[pl.ds(off, CHUNK)], idxd[b], isems_d[b])

    def _issue_gather(b):
        pltpu.make_async_copy(src3.at[pl.ds(0, CHUNK)], idxs[b],
                              isems_s[b]).wait()
        pltpu.async_copy(table.at[idxs[b]], rowbufs[b], gsems[b])

    def _issue_scatter(b):
        pltpu.make_async_copy(table.at[idxs[b]], rowbufs[b], gsems[b]).wait()
        pltpu.make_async_copy(dst3.at[pl.ds(0, CHUNK)], idxd[b],
                              isems_d[b]).wait()
        pltpu.async_copy(rowbufs[b], acc.at[idxd[b]], ssems[b], add=True)

    def group(q, carry):
        base = wid * EPW + q * NB * CHUNK
        for b in (0, 1):
            @pl.when(q > 0)
            def _(b=b):
                _wait_prev_scatter(b)
            _issue_idx(base, b)
        for b in (0, 1):
            _issue_gather(b)
        for b in (2, 3):
            @pl.when(q > 0)
            def _(b=b):
                _wait_prev_scatter(b)
            _issue_idx(base, b)
        for b in (0, 1):
            _issue_scatter(b)
        for b in (2, 3):
            _issue_gather(b)
        for b in (2, 3):
            _issue_scatter(b)
        return carry

    lax.fori_loop(0, NGROUPS, group, 0)
    for b in range(NB):
        _wait_prev_scatter(b)

    if ETAIL:
        toff = wid * EPW + NCHUNKS * CHUNK
        pltpu.sync_copy(src3.at[pl.ds(toff, ETAIL)], tidx_s)
        pltpu.sync_copy(dst3.at[pl.ds(toff, ETAIL)], tidx_d)
        pltpu.async_copy(table.at[tidx_s], rowbufs[0].at[pl.ds(0, ETAIL)],
                         gsems[0]).wait()
        pltpu.sync_copy(rowbufs[0].at[pl.ds(0, ETAIL)], acc.at[tidx_d],
                        add=True)

    plsc.subcore_barrier()

    @pl.when(sid < NS - 1)
    def _():
        pltpu.sync_copy(acc.at[pl.ds(r0, RPS)], out.at[cid, pl.ds(r0, RPS)])

    @pl.when(sid == NS - 1)
    def _():
        last = (NS - 1) * RPS
        pltpu.sync_copy(acc.at[pl.ds(last, RPS_LAST)],
                        out.at[cid, pl.ds(last, RPS_LAST)])


_agg = pl.kernel(
    _agg_entry,
    out_type=jax.ShapeDtypeStruct((NC, N, H), jnp.float32),
    mesh=plsc.VectorSubcoreMesh(core_axis_name="c", subcore_axis_name="s"),
    scratch_types=(
        [pltpu.VMEM((CHUNK,), jnp.int32) for _ in range(2 * NB)]
        + [pltpu.VMEM((CHUNK, H), jnp.float32) for _ in range(NB)]
        + [pltpu.VMEM((ETAIL,), jnp.int32) for _ in range(2)]
        + [pltpu.VMEM_SHARED((N, H), jnp.float32)]
        + [pltpu.SemaphoreType.DMA for _ in range(4 * NB)]
    ),
)


def _mlp_body(p_ref, w_ref, b_ref, o_ref, *, relu):
    z = p_ref[0] + p_ref[1]
    y = jnp.dot(z, w_ref[...], preferred_element_type=jnp.float32) + b_ref[...]
    if relu:
        y = jnp.maximum(y, 0.0)
    o_ref[...] = y


def _mlp(p, w, b, relu):
    bn = 2000
    return pl.pallas_call(
        functools.partial(_mlp_body, relu=relu),
        grid=(N // bn,),
        in_specs=[
            pl.BlockSpec((NC, bn, H), lambda i: (0, i, 0)),
            pl.BlockSpec((H, H), lambda i: (0, 0)),
            pl.BlockSpec((1, H), lambda i: (0, 0)),
        ],
        out_specs=pl.BlockSpec((bn, H), lambda i: (i, 0)),
        out_shape=jax.ShapeDtypeStruct((N, H), jnp.float32),
    )(p, w, b.reshape(1, H))


def kernel(x, edge_index, W0, b0, bn_gamma, bn_beta, W1, b1):
    src = edge_index[0]
    dst = edge_index[1]
    zeros = jnp.zeros((N, H), jnp.float32)
    s = bn_gamma * jax.lax.rsqrt(1.0 + 1e-5)
    W0p = W0.T * s[None, :]
    b0p = b0 * s + bn_beta

    p0 = _agg(x, src, dst, zeros)
    h = _mlp(p0, W0p, b0p, relu=True)
    p1 = _agg(h, src, dst, zeros)
    out = _mlp(p1, W1.T, b1, relu=False)
    return out
